# baseline (speedup 1.0000x reference)
import functools

import numpy as np
import jax
import jax.numpy as jnp
from jax import lax
from jax.experimental import pallas as pl
from jax.experimental.pallas import tpu as pltpu
from jax.experimental.pallas import tpu_sc as plsc

B = 32
D = 256
SLOTS = 4096
INV_SQRT_D = 1.0 / 16.0

NC = 2
NS = 16
LANES = 16
NQ = D // LANES
CH = 128
NCH = SLOTS // CH
GRP = 16
NG = SLOTS // LANES


def _pre_kernel(s_ref, wvec_ref, gate_ref, Wq_ref, Wl_ref, bl_ref, Wk_ref,
                Wv_ref, gw_ref, q_ref, wval_ref, c_ref):
    s = s_ref[...]
    logits = jax.lax.dot_general(s, Wl_ref[...], (((1,), (1,)), ((), ())),
                                 preferred_element_type=jnp.float32)
    logits = logits + bl_ref[...][None, :]
    m = jnp.max(logits, axis=-1, keepdims=True)
    e = jnp.exp(logits - m)
    w = e / jnp.sum(e, axis=-1, keepdims=True)
    gw_ref[...] = gate_ref[...] * w
    q = jax.lax.dot_general(s, Wq_ref[...], (((1,), (1,)), ((), ())),
                            preferred_element_type=jnp.float32)
    q_ref[...] = q
    wvec = wvec_ref[...]
    wk = jax.lax.dot_general(wvec, Wk_ref[...], (((1,), (1,)), ((), ())),
                             preferred_element_type=jnp.float32)
    wval_ref[...] = jax.lax.dot_general(wvec, Wv_ref[...],
                                        (((1,), (1,)), ((), ())),
                                        preferred_element_type=jnp.float32)
    c = jnp.sum(q * wk, axis=-1, keepdims=True)
    c_ref[...] = jnp.broadcast_to(c, (B, 128))


def _sc_body(memk_ref, memv_ref, gw_ref, q_ref, wval_ref, c_ref, out_ref,
             kbuf0, kbuf1, gw_v, att_v, q_v, wval_v, c_v, out_v, sem0, sem1):
    cid = lax.axis_index("c")
    sid = lax.axis_index("s")
    wid = sid * NC + cid
    base = wid * SLOTS

    pltpu.sync_copy(gw_ref.at[pl.ds(base, SLOTS)], gw_v)
    pltpu.sync_copy(q_ref.at[pl.ds(wid * D, D)], q_v)
    pltpu.sync_copy(wval_ref.at[pl.ds(wid * D, D)], wval_v)
    pltpu.sync_copy(c_ref.at[pl.ds(wid * 128, LANES)], c_v)

    qs = [q_v[pl.ds(LANES * j, LANES)] for j in range(NQ)]
    c_spl = c_v[...]
    lane = lax.iota(jnp.int32, LANES)

    def stream_pass(src_ref, process):
        def start(ci, dst, sem):
            src = src_ref.at[pl.ds((base + ci * CH) * D, CH * D)]
            pltpu.make_async_copy(src, dst, sem).start()

        def wait(dst, sem):
            src = src_ref.at[pl.ds(base * D, CH * D)]
            pltpu.make_async_copy(src, dst, sem).wait()

        start(0, kbuf0, sem0)

        def body(p, _):
            c0 = 2 * p
            c1 = 2 * p + 1
            start(c1, kbuf1, sem1)
            wait(kbuf0, sem0)
            process(c0, kbuf0)

            @pl.when(c1 + 1 < NCH)
            def _():
                start(c1 + 1, kbuf0, sem0)

            wait(kbuf1, sem1)
            process(c1, kbuf1)
            return 0

        lax.fori_loop(0, NCH // 2, body, 0)

    def att_chunk(cidx, buf):
        def grp_body(g, _):
            att16 = jnp.zeros((LANES,), jnp.float32)
            for j in range(GRP):
                off = (g * GRP + j) * D
                acc = [buf[pl.ds(off + m * LANES, LANES)] * qs[m]
                       for m in range(4)]
                for m in range(4, NQ):
                    acc[m % 4] = acc[m % 4] + \
                        buf[pl.ds(off + m * LANES, LANES)] * qs[m]
                dot = jnp.sum((acc[0] + acc[1]) + (acc[2] + acc[3]))
                att16 = jnp.where(lane == j, dot, att16)
            pos = cidx * CH + g * GRP
            gw16 = gw_v[pl.ds(pos, LANES)]
            att_v[pl.ds(pos, LANES)] = (
                (att16 * (1.0 - gw16) + gw16 * c_spl) * INV_SQRT_D)
            return 0

        lax.fori_loop(0, CH // GRP, grp_body, 0)

    stream_pass(memk_ref, att_chunk)

    def max_body(i, m16):
        return jnp.maximum(m16, att_v[pl.ds(i * LANES, LANES)])

    m16 = lax.fori_loop(0, NG, max_body,
                        jnp.full((LANES,), -1e30, jnp.float32))
    m = jnp.max(m16)

    def exp_body(i, carry):
        s16, sg16 = carry
        a = att_v[pl.ds(i * LANES, LANES)]
        gw16 = gw_v[pl.ds(i * LANES, LANES)]
        e = jnp.exp(a - m)
        att_v[pl.ds(i * LANES, LANES)] = e * (1.0 - gw16)
        return (s16 + e, sg16 + e * gw16)

    zero16 = jnp.zeros((LANES,), jnp.float32)
    s16, sg16 = lax.fori_loop(0, NG, exp_body, (zero16, zero16))
    denom = jnp.sum(s16)
    sgw = jnp.sum(sg16)

    for mm in range(NQ):
        out_v[pl.ds(mm * LANES, LANES)] = zero16

    def out_chunk(cidx, buf):
        def grp_body(g, _):
            pos = cidx * CH + g * GRP
            w16 = att_v[pl.ds(pos, LANES)]
            accs = [out_v[pl.ds(mm * LANES, LANES)] for mm in range(NQ)]
            for j in range(GRP):
                off = (g * GRP + j) * D
                ws = jnp.sum(jnp.where(lane == j, w16, 0.0))
                for mm in range(NQ):
                    accs[mm] = accs[mm] + \
                        ws * buf[pl.ds(off + mm * LANES, LANES)]
            for mm in range(NQ):
                out_v[pl.ds(mm * LANES, LANES)] = accs[mm]
            return 0

        lax.fori_loop(0, CH // GRP, grp_body, 0)

    stream_pass(memv_ref, out_chunk)

    dvec = jnp.broadcast_to(denom, (LANES,))
    for mm in range(NQ):
        r = (out_v[pl.ds(mm * LANES, LANES)]
             + sgw * wval_v[pl.ds(mm * LANES, LANES)]) / dvec
        out_v[pl.ds(mm * LANES, LANES)] = r
    pltpu.sync_copy(out_v, out_ref.at[pl.ds(wid * D, D)])


_sc_call = functools.partial(
    pl.kernel,
    out_type=jax.ShapeDtypeStruct((B * D,), jnp.float32),
    mesh=plsc.VectorSubcoreMesh(core_axis_name="c", subcore_axis_name="s"),
    compiler_params=pltpu.CompilerParams(needs_layout_passes=False),
    scratch_types=[
        pltpu.VMEM((CH * D,), jnp.float32),
        pltpu.VMEM((CH * D,), jnp.float32),
        pltpu.VMEM((SLOTS,), jnp.float32),
        pltpu.VMEM((SLOTS,), jnp.float32),
        pltpu.VMEM((D,), jnp.float32),
        pltpu.VMEM((D,), jnp.float32),
        pltpu.VMEM((LANES,), jnp.float32),
        pltpu.VMEM((D,), jnp.float32),
        pltpu.SemaphoreType.DMA,
        pltpu.SemaphoreType.DMA,
    ],
)(_sc_body)


def kernel(s, write_vec, mem_k, mem_v, gate, Wq, Wl, bl, Wk, Wv):
    f32 = jnp.float32
    gw, q, wval, c = pl.pallas_call(
        _pre_kernel,
        out_shape=(
            jax.ShapeDtypeStruct((B, SLOTS), f32),
            jax.ShapeDtypeStruct((B, D), f32),
            jax.ShapeDtypeStruct((B, D), f32),
            jax.ShapeDtypeStruct((B, 128), f32),
        ),
    )(s, write_vec, gate, Wq, Wl, bl, Wk, Wv)

    out = _sc_call(
        mem_k.reshape(B * SLOTS * D),
        mem_v.reshape(B * SLOTS * D),
        gw.reshape(B * SLOTS),
        q.reshape(B * D),
        wval.reshape(B * D),
        c.reshape(B * 128),
    )
    return out.reshape(B, D)


# speedup vs baseline: 1.7073x; 1.7073x over previous
import functools

import numpy as np
import jax
import jax.numpy as jnp
from jax import lax
from jax.experimental import pallas as pl
from jax.experimental.pallas import tpu as pltpu
from jax.experimental.pallas import tpu_sc as plsc

B = 32
D = 256
SLOTS = 4096
INV_SQRT_D = 1.0 / 16.0

NC = 2
NS = 16
LANES = 16
NQ = D // LANES
CH = 128
NCH = SLOTS // CH
GRP = 16
NG = SLOTS // LANES


def _pre_kernel(s_ref, wvec_ref, gate_ref, Wq_ref, Wl_ref, bl_ref, Wk_ref,
                Wv_ref, gw_ref, q_ref, wval_ref, c_ref):
    s = s_ref[...]
    logits = jax.lax.dot_general(s, Wl_ref[...], (((1,), (1,)), ((), ())),
                                 preferred_element_type=jnp.float32)
    logits = logits + bl_ref[...][None, :]
    m = jnp.max(logits, axis=-1, keepdims=True)
    e = jnp.exp(logits - m)
    w = e / jnp.sum(e, axis=-1, keepdims=True)
    gw_ref[...] = gate_ref[...] * w
    q = jax.lax.dot_general(s, Wq_ref[...], (((1,), (1,)), ((), ())),
                            preferred_element_type=jnp.float32)
    q_ref[...] = q
    wvec = wvec_ref[...]
    wk = jax.lax.dot_general(wvec, Wk_ref[...], (((1,), (1,)), ((), ())),
                             preferred_element_type=jnp.float32)
    wval_ref[...] = jax.lax.dot_general(wvec, Wv_ref[...],
                                        (((1,), (1,)), ((), ())),
                                        preferred_element_type=jnp.float32)
    c = jnp.sum(q * wk, axis=-1, keepdims=True)
    c_ref[...] = jnp.broadcast_to(c, (B, 128))


def _sc_body(memk_ref, memv_ref, gw_ref, q_ref, wval_ref, c_ref, out_ref,
             kbuf0, kbuf1, gw_v, att_v, q_v, wval_v, c_v, out_v, sem0, sem1):
    cid = lax.axis_index("c")
    sid = lax.axis_index("s")
    wid = sid * NC + cid
    base = wid * SLOTS

    pltpu.sync_copy(gw_ref.at[pl.ds(base, SLOTS)], gw_v)
    pltpu.sync_copy(q_ref.at[pl.ds(wid * D, D)], q_v)
    pltpu.sync_copy(wval_ref.at[pl.ds(wid * D, D)], wval_v)
    pltpu.sync_copy(c_ref.at[pl.ds(wid * 128, LANES)], c_v)

    qs = [q_v[pl.ds(LANES * j, LANES)] for j in range(NQ)]
    c_spl = c_v[...]
    lane = lax.iota(jnp.int32, LANES)

    def stream_pass(src_ref, process):
        def start(ci, dst, sem):
            src = src_ref.at[pl.ds(base + ci * CH, CH)]
            pltpu.make_async_copy(src, dst, sem).start()

        def wait(dst, sem):
            src = src_ref.at[pl.ds(base, CH)]
            pltpu.make_async_copy(src, dst, sem).wait()

        start(0, kbuf0, sem0)

        def body(p, _):
            c0 = 2 * p
            c1 = 2 * p + 1
            start(c1, kbuf1, sem1)
            wait(kbuf0, sem0)
            process(c0, kbuf0)

            @pl.when(c1 + 1 < NCH)
            def _():
                start(c1 + 1, kbuf0, sem0)

            wait(kbuf1, sem1)
            process(c1, kbuf1)
            return 0

        lax.fori_loop(0, NCH // 2, body, 0)

    def att_chunk(cidx, buf):
        def grp_body(g, _):
            att16 = jnp.zeros((LANES,), jnp.float32)
            for j in range(GRP):
                row = g * GRP + j
                acc = [buf[row, pl.ds(m * LANES, LANES)] * qs[m]
                       for m in range(4)]
                for m in range(4, NQ):
                    acc[m % 4] = acc[m % 4] + \
                        buf[row, pl.ds(m * LANES, LANES)] * qs[m]
                dot = jnp.sum((acc[0] + acc[1]) + (acc[2] + acc[3]))
                att16 = jnp.where(lane == j, dot, att16)
            pos = cidx * CH + g * GRP
            gw16 = gw_v[pl.ds(pos, LANES)]
            att_v[pl.ds(pos, LANES)] = (
                (att16 * (1.0 - gw16) + gw16 * c_spl) * INV_SQRT_D)
            return 0

        lax.fori_loop(0, CH // GRP, grp_body, 0)

    stream_pass(memk_ref, att_chunk)

    def max_body(i, m16):
        return jnp.maximum(m16, att_v[pl.ds(i * LANES, LANES)])

    m16 = lax.fori_loop(0, NG, max_body,
                        jnp.full((LANES,), -1e30, jnp.float32))
    m = jnp.max(m16)

    def exp_body(i, carry):
        s16, sg16 = carry
        a = att_v[pl.ds(i * LANES, LANES)]
        gw16 = gw_v[pl.ds(i * LANES, LANES)]
        e = jnp.exp(a - m)
        att_v[pl.ds(i * LANES, LANES)] = e * (1.0 - gw16)
        return (s16 + e, sg16 + e * gw16)

    zero16 = jnp.zeros((LANES,), jnp.float32)
    s16, sg16 = lax.fori_loop(0, NG, exp_body, (zero16, zero16))
    denom = jnp.sum(s16)
    sgw = jnp.sum(sg16)

    for mm in range(NQ):
        out_v[pl.ds(mm * LANES, LANES)] = zero16

    def out_chunk(cidx, buf):
        def grp_body(g, _):
            pos = cidx * CH + g * GRP
            w16 = att_v[pl.ds(pos, LANES)]
            accs = [out_v[pl.ds(mm * LANES, LANES)] for mm in range(NQ)]
            for j in range(GRP):
                row = g * GRP + j
                ws = jnp.sum(jnp.where(lane == j, w16, 0.0))
                for mm in range(NQ):
                    accs[mm] = accs[mm] + \
                        ws * buf[row, pl.ds(mm * LANES, LANES)]
            for mm in range(NQ):
                out_v[pl.ds(mm * LANES, LANES)] = accs[mm]
            return 0

        lax.fori_loop(0, CH // GRP, grp_body, 0)

    stream_pass(memv_ref, out_chunk)

    dvec = jnp.broadcast_to(denom, (LANES,))
    for mm in range(NQ):
        r = (out_v[pl.ds(mm * LANES, LANES)]
             + sgw * wval_v[pl.ds(mm * LANES, LANES)]) / dvec
        out_v[pl.ds(mm * LANES, LANES)] = r
    pltpu.sync_copy(out_v, out_ref.at[pl.ds(wid * D, D)])


_sc_call = functools.partial(
    pl.kernel,
    out_type=jax.ShapeDtypeStruct((B * D,), jnp.float32),
    mesh=plsc.VectorSubcoreMesh(core_axis_name="c", subcore_axis_name="s"),
    compiler_params=pltpu.CompilerParams(needs_layout_passes=False),
    scratch_types=[
        pltpu.VMEM((CH, D), jnp.float32),
        pltpu.VMEM((CH, D), jnp.float32),
        pltpu.VMEM((SLOTS,), jnp.float32),
        pltpu.VMEM((SLOTS,), jnp.float32),
        pltpu.VMEM((D,), jnp.float32),
        pltpu.VMEM((D,), jnp.float32),
        pltpu.VMEM((LANES,), jnp.float32),
        pltpu.VMEM((D,), jnp.float32),
        pltpu.SemaphoreType.DMA,
        pltpu.SemaphoreType.DMA,
    ],
)(_sc_body)


def kernel(s, write_vec, mem_k, mem_v, gate, Wq, Wl, bl, Wk, Wv):
    f32 = jnp.float32
    gw, q, wval, c = pl.pallas_call(
        _pre_kernel,
        out_shape=(
            jax.ShapeDtypeStruct((B, SLOTS), f32),
            jax.ShapeDtypeStruct((B, D), f32),
            jax.ShapeDtypeStruct((B, D), f32),
            jax.ShapeDtypeStruct((B, 128), f32),
        ),
    )(s, write_vec, gate, Wq, Wl, bl, Wk, Wv)

    out = _sc_call(
        mem_k.reshape(B * SLOTS, D),
        mem_v.reshape(B * SLOTS, D),
        gw.reshape(B * SLOTS),
        q.reshape(B * D),
        wval.reshape(B * D),
        c.reshape(B * 128),
    )
    return out.reshape(B, D)


# speedup vs baseline: 2.7741x; 1.6249x over previous
import functools

import numpy as np
import jax
import jax.numpy as jnp
from jax import lax
from jax.experimental import pallas as pl
from jax.experimental.pallas import tpu as pltpu
from jax.experimental.pallas import tpu_sc as plsc

B = 32
D = 256
SLOTS = 4096
INV_SQRT_D = 1.0 / 16.0

NC = 2
NS = 16
LANES = 16
NQ = D // LANES
CH = 128
NCH = SLOTS // CH
GRP = 16
NG = SLOTS // LANES


def _pre_kernel(s_ref, wvec_ref, gate_ref, Wq_ref, Wl_ref, bl_ref, Wk_ref,
                Wv_ref, gw_ref, q_ref, wval_ref, c_ref):
    s = s_ref[...]
    logits = jax.lax.dot_general(s, Wl_ref[...], (((1,), (1,)), ((), ())),
                                 preferred_element_type=jnp.float32)
    logits = logits + bl_ref[...][None, :]
    m = jnp.max(logits, axis=-1, keepdims=True)
    e = jnp.exp(logits - m)
    w = e / jnp.sum(e, axis=-1, keepdims=True)
    gw_ref[...] = gate_ref[...] * w
    q = jax.lax.dot_general(s, Wq_ref[...], (((1,), (1,)), ((), ())),
                            preferred_element_type=jnp.float32)
    q_ref[...] = q
    wvec = wvec_ref[...]
    wk = jax.lax.dot_general(wvec, Wk_ref[...], (((1,), (1,)), ((), ())),
                             preferred_element_type=jnp.float32)
    wval_ref[...] = jax.lax.dot_general(wvec, Wv_ref[...],
                                        (((1,), (1,)), ((), ())),
                                        preferred_element_type=jnp.float32)
    c = jnp.sum(q * wk, axis=-1, keepdims=True)
    c_ref[...] = jnp.broadcast_to(c, (B, 128))


def _sc_body(memk_ref, memv_ref, gw_ref, q_ref, wval_ref, c_ref, out_ref,
             kbuf0, kbuf1, gw_v, att_v, q_v, wval_v, c_v, out_v, sem0, sem1):
    cid = lax.axis_index("c")
    sid = lax.axis_index("s")
    wid = sid * NC + cid
    base = wid * SLOTS

    pltpu.sync_copy(gw_ref.at[pl.ds(base, SLOTS)], gw_v)
    pltpu.sync_copy(q_ref.at[pl.ds(wid * D, D)], q_v)
    pltpu.sync_copy(wval_ref.at[pl.ds(wid * D, D)], wval_v)
    pltpu.sync_copy(c_ref.at[pl.ds(wid * 128, LANES)], c_v)

    c_spl = c_v[...]
    lane = lax.iota(jnp.int32, LANES)

    def stream_pass(src_ref, process):
        def start(ci, dst, sem):
            src = src_ref.at[pl.ds(base + ci * CH, CH)]
            pltpu.make_async_copy(src, dst, sem).start()

        def wait(dst, sem):
            src = src_ref.at[pl.ds(base, CH)]
            pltpu.make_async_copy(src, dst, sem).wait()

        start(0, kbuf0, sem0)

        def body(p, _):
            c0 = 2 * p
            c1 = 2 * p + 1
            start(c1, kbuf1, sem1)
            wait(kbuf0, sem0)
            process(c0, kbuf0)

            @pl.when(c1 + 1 < NCH)
            def _():
                start(c1 + 1, kbuf0, sem0)

            wait(kbuf1, sem1)
            process(c1, kbuf1)
            return 0

        lax.fori_loop(0, NCH // 2, body, 0)

    def att_chunk(cidx, buf):
        def grp_body(g, _):
            prod = [jnp.zeros((LANES,), jnp.float32) for _ in range(GRP)]
            for m in range(NQ):
                qm = q_v[pl.ds(m * LANES, LANES)]
                for j in range(GRP):
                    row = g * GRP + j
                    prod[j] = prod[j] + qm * buf[row, pl.ds(m * LANES, LANES)]
            att16 = jnp.zeros((LANES,), jnp.float32)
            for j in range(GRP):
                att16 = jnp.where(lane == j, jnp.sum(prod[j]), att16)
            pos = cidx * CH + g * GRP
            gw16 = gw_v[pl.ds(pos, LANES)]
            att_v[pl.ds(pos, LANES)] = (
                (att16 * (1.0 - gw16) + gw16 * c_spl) * INV_SQRT_D)
            return 0

        lax.fori_loop(0, CH // GRP, grp_body, 0)

    stream_pass(memk_ref, att_chunk)

    def max_body(i, m16):
        return jnp.maximum(m16, att_v[pl.ds(i * LANES, LANES)])

    m16 = lax.fori_loop(0, NG, max_body,
                        jnp.full((LANES,), -1e30, jnp.float32))
    m = jnp.max(m16)

    def exp_body(i, carry):
        s16, sg16 = carry
        a = att_v[pl.ds(i * LANES, LANES)]
        gw16 = gw_v[pl.ds(i * LANES, LANES)]
        e = jnp.exp(a - m)
        att_v[pl.ds(i * LANES, LANES)] = e * (1.0 - gw16)
        return (s16 + e, sg16 + e * gw16)

    zero16 = jnp.zeros((LANES,), jnp.float32)
    s16, sg16 = lax.fori_loop(0, NG, exp_body, (zero16, zero16))
    denom = jnp.sum(s16)
    sgw = jnp.sum(sg16)

    for mm in range(NQ):
        out_v[pl.ds(mm * LANES, LANES)] = zero16

    def out_chunk(cidx, buf):
        def grp_body(g, _):
            pos = cidx * CH + g * GRP
            w16 = att_v[pl.ds(pos, LANES)]
            splats = [
                w16.at[jnp.full((LANES,), j, jnp.int32)].get(
                    mode="promise_in_bounds")
                for j in range(GRP)
            ]
            for h in range(2):
                accs = [out_v[pl.ds((8 * h + mm) * LANES, LANES)]
                        for mm in range(8)]
                for j in range(GRP):
                    row = g * GRP + j
                    for mm in range(8):
                        accs[mm] = accs[mm] + splats[j] * \
                            buf[row, pl.ds((8 * h + mm) * LANES, LANES)]
                for mm in range(8):
                    out_v[pl.ds((8 * h + mm) * LANES, LANES)] = accs[mm]
            return 0

        lax.fori_loop(0, CH // GRP, grp_body, 0)

    stream_pass(memv_ref, out_chunk)

    dvec = jnp.broadcast_to(denom, (LANES,))
    for mm in range(NQ):
        r = (out_v[pl.ds(mm * LANES, LANES)]
             + sgw * wval_v[pl.ds(mm * LANES, LANES)]) / dvec
        out_v[pl.ds(mm * LANES, LANES)] = r
    pltpu.sync_copy(out_v, out_ref.at[pl.ds(wid * D, D)])


_sc_call = functools.partial(
    pl.kernel,
    out_type=jax.ShapeDtypeStruct((B * D,), jnp.float32),
    mesh=plsc.VectorSubcoreMesh(core_axis_name="c", subcore_axis_name="s"),
    compiler_params=pltpu.CompilerParams(needs_layout_passes=False),
    scratch_types=[
        pltpu.VMEM((CH, D), jnp.float32),
        pltpu.VMEM((CH, D), jnp.float32),
        pltpu.VMEM((SLOTS,), jnp.float32),
        pltpu.VMEM((SLOTS,), jnp.float32),
        pltpu.VMEM((D,), jnp.float32),
        pltpu.VMEM((D,), jnp.float32),
        pltpu.VMEM((LANES,), jnp.float32),
        pltpu.VMEM((D,), jnp.float32),
        pltpu.SemaphoreType.DMA,
        pltpu.SemaphoreType.DMA,
    ],
)(_sc_body)


def kernel(s, write_vec, mem_k, mem_v, gate, Wq, Wl, bl, Wk, Wv):
    f32 = jnp.float32
    gw, q, wval, c = pl.pallas_call(
        _pre_kernel,
        out_shape=(
            jax.ShapeDtypeStruct((B, SLOTS), f32),
            jax.ShapeDtypeStruct((B, D), f32),
            jax.ShapeDtypeStruct((B, D), f32),
            jax.ShapeDtypeStruct((B, 128), f32),
        ),
    )(s, write_vec, gate, Wq, Wl, bl, Wk, Wv)

    out = _sc_call(
        mem_k.reshape(B * SLOTS, D),
        mem_v.reshape(B * SLOTS, D),
        gw.reshape(B * SLOTS),
        q.reshape(B * D),
        wval.reshape(B * D),
        c.reshape(B * 128),
    )
    return out.reshape(B, D)


# speedup vs baseline: 3.7947x; 1.3679x over previous
import functools

import jax
import jax.numpy as jnp
from jax import lax
from jax.experimental import pallas as pl
from jax.experimental.pallas import tpu as pltpu
from jax.experimental.pallas import tpu_sc as plsc

B = 32
D = 256
SLOTS = 4096
INV_SQRT_D = 1.0 / 16.0

T = 2304
SBLK = 256
NBT = T // SBLK

NC = 2
NS = 16
LANES = 16
NQ = D // LANES
CH = 128
L_SC = SLOTS - T
NCH = L_SC // CH
GRP = 16
NG_SC = L_SC // LANES


def _pre_kernel(s_ref, wvec_ref, gate_ref, Wq_ref, Wl_ref, bl_ref, Wk_ref,
                Wv_ref, gw_ref, q_ref, wval_ref, c_ref):
    s = s_ref[...]
    logits = jax.lax.dot_general(s, Wl_ref[...], (((1,), (1,)), ((), ())),
                                 preferred_element_type=jnp.float32)
    logits = logits + bl_ref[...][None, :]
    m = jnp.max(logits, axis=-1, keepdims=True)
    e = jnp.exp(logits - m)
    w = e / jnp.sum(e, axis=-1, keepdims=True)
    gw_ref[...] = gate_ref[...] * w
    q = jax.lax.dot_general(s, Wq_ref[...], (((1,), (1,)), ((), ())),
                            preferred_element_type=jnp.float32)
    q_ref[...] = q
    wvec = wvec_ref[...]
    wk = jax.lax.dot_general(wvec, Wk_ref[...], (((1,), (1,)), ((), ())),
                             preferred_element_type=jnp.float32)
    wval_ref[...] = jax.lax.dot_general(wvec, Wv_ref[...],
                                        (((1,), (1,)), ((), ())),
                                        preferred_element_type=jnp.float32)
    c = jnp.sum(q * wk, axis=-1, keepdims=True)
    c_ref[...] = jnp.broadcast_to(c, (B, 128))


def _tc_main(q_ref, gw_ref, c_ref, mk_ref, mv_ref,
             o1_ref, m1_ref, l1_ref, sg1_ref, att_s, acc_s):
    g = pl.program_id(0)

    @pl.when(g < NBT)
    def _att_phase():
        i = g
        q = q_ref[...]
        gw = gw_ref[pl.ds(0, B), pl.ds(i * SBLK, SBLK)]
        c = c_ref[...][:, :1]
        a0 = jnp.sum(q[:, None, :] * mk_ref[...], axis=-1)
        att_s[pl.ds(0, B), pl.ds(i * SBLK, SBLK)] = (
            (a0 * (1.0 - gw) + gw * c) * INV_SQRT_D)

    @pl.when(g == NBT)
    def _softmax_partial():
        att = att_s[...]
        gw = gw_ref[pl.ds(0, B), pl.ds(0, T)]
        m1 = jnp.max(att, axis=-1, keepdims=True)
        e = jnp.exp(att - m1)
        att_s[...] = e * (1.0 - gw)
        m1_ref[...] = jnp.broadcast_to(m1, (B, 128))
        l1_ref[...] = jnp.broadcast_to(
            jnp.sum(e, axis=-1, keepdims=True), (B, 128))
        sg1_ref[...] = jnp.broadcast_to(
            jnp.sum(e * gw, axis=-1, keepdims=True), (B, 128))
        acc_s[...] = jnp.zeros((B, D), jnp.float32)

    @pl.when(g >= NBT)
    def _out_phase():
        i = g - NBT
        wr = att_s[pl.ds(0, B), pl.ds(i * SBLK, SBLK)]
        acc_s[...] += jnp.sum(wr[:, :, None] * mv_ref[...], axis=1)

    @pl.when(g == 2 * NBT - 1)
    def _epilogue():
        o1_ref[...] = acc_s[...]


def _sc_body(memk_ref, memv_ref, gw_ref, q_ref, c_ref,
             o2_ref, m2_ref, l2_ref, sg2_ref,
             kbuf0, kbuf1, gw_v, att_v, q_v, c_v, out_v, stat_v, sem0, sem1):
    cid = lax.axis_index("c")
    sid = lax.axis_index("s")
    wid = sid * NC + cid
    base = wid * SLOTS + T

    pltpu.sync_copy(gw_ref.at[pl.ds(base, L_SC)], gw_v)
    pltpu.sync_copy(q_ref.at[pl.ds(wid * D, D)], q_v)
    pltpu.sync_copy(c_ref.at[pl.ds(wid * 128, LANES)], c_v)

    c_spl = c_v[...]
    lane = lax.iota(jnp.int32, LANES)

    def stream_pass(src_ref, process):
        def start(ci, dst, sem):
            src = src_ref.at[pl.ds(base + ci * CH, CH)]
            pltpu.make_async_copy(src, dst, sem).start()

        def wait(dst, sem):
            src = src_ref.at[pl.ds(base, CH)]
            pltpu.make_async_copy(src, dst, sem).wait()

        start(0, kbuf0, sem0)

        def body(p, _):
            c1 = 2 * p + 1
            start(c1, kbuf1, sem1)
            wait(kbuf0, sem0)
            process(2 * p, kbuf0)

            @pl.when(c1 + 1 < NCH)
            def _():
                start(c1 + 1, kbuf0, sem0)

            wait(kbuf1, sem1)
            process(c1, kbuf1)
            return 0

        lax.fori_loop(0, NCH // 2, body, 0)

    def att_chunk(cidx, buf):
        def grp_body(g, _):
            prod = [jnp.zeros((LANES,), jnp.float32) for _ in range(GRP)]
            for m in range(NQ):
                qm = q_v[pl.ds(m * LANES, LANES)]
                for j in range(GRP):
                    row = g * GRP + j
                    prod[j] = prod[j] + qm * buf[row, pl.ds(m * LANES, LANES)]
            att16 = jnp.zeros((LANES,), jnp.float32)
            for j in range(GRP):
                att16 = jnp.where(lane == j, jnp.sum(prod[j]), att16)
            pos = cidx * CH + g * GRP
            gw16 = gw_v[pl.ds(pos, LANES)]
            att_v[pl.ds(pos, LANES)] = (
                (att16 * (1.0 - gw16) + gw16 * c_spl) * INV_SQRT_D)
            return 0

        lax.fori_loop(0, CH // GRP, grp_body, 0)

    stream_pass(memk_ref, att_chunk)

    def max_body(i, m16):
        return jnp.maximum(m16, att_v[pl.ds(i * LANES, LANES)])

    m16 = lax.fori_loop(0, NG_SC, max_body,
                        jnp.full((LANES,), -1e30, jnp.float32))
    m = jnp.max(m16)

    def exp_body(i, carry):
        s16, sg16 = carry
        a = att_v[pl.ds(i * LANES, LANES)]
        gw16 = gw_v[pl.ds(i * LANES, LANES)]
        e = jnp.exp(a - m)
        att_v[pl.ds(i * LANES, LANES)] = e * (1.0 - gw16)
        return (s16 + e, sg16 + e * gw16)

    zero16 = jnp.zeros((LANES,), jnp.float32)
    s16, sg16 = lax.fori_loop(0, NG_SC, exp_body, (zero16, zero16))
    l2 = jnp.sum(s16)
    sg2 = jnp.sum(sg16)

    for mm in range(NQ):
        out_v[pl.ds(mm * LANES, LANES)] = zero16

    def out_chunk(cidx, buf):
        def grp_body(g, _):
            pos = cidx * CH + g * GRP
            w16 = att_v[pl.ds(pos, LANES)]
            splats = [
                w16.at[jnp.full((LANES,), j, jnp.int32)].get(
                    mode="promise_in_bounds")
                for j in range(GRP)
            ]
            for h in range(2):
                accs = [out_v[pl.ds((8 * h + mm) * LANES, LANES)]
                        for mm in range(8)]
                for j in range(GRP):
                    row = g * GRP + j
                    for mm in range(8):
                        accs[mm] = accs[mm] + splats[j] * \
                            buf[row, pl.ds((8 * h + mm) * LANES, LANES)]
                for mm in range(8):
                    out_v[pl.ds((8 * h + mm) * LANES, LANES)] = accs[mm]
            return 0

        lax.fori_loop(0, CH // GRP, grp_body, 0)

    stream_pass(memv_ref, out_chunk)

    pltpu.sync_copy(out_v, o2_ref.at[pl.ds(wid * D, D)])
    stat_v[...] = jnp.broadcast_to(m, (LANES,))
    pltpu.sync_copy(stat_v, m2_ref.at[pl.ds(wid * LANES, LANES)])
    stat_v[...] = jnp.broadcast_to(l2, (LANES,))
    pltpu.sync_copy(stat_v, l2_ref.at[pl.ds(wid * LANES, LANES)])
    stat_v[...] = jnp.broadcast_to(sg2, (LANES,))
    pltpu.sync_copy(stat_v, sg2_ref.at[pl.ds(wid * LANES, LANES)])


_sc_call = functools.partial(
    pl.kernel,
    out_type=(
        jax.ShapeDtypeStruct((B * D,), jnp.float32),
        jax.ShapeDtypeStruct((B * LANES,), jnp.float32),
        jax.ShapeDtypeStruct((B * LANES,), jnp.float32),
        jax.ShapeDtypeStruct((B * LANES,), jnp.float32),
    ),
    mesh=plsc.VectorSubcoreMesh(core_axis_name="c", subcore_axis_name="s"),
    compiler_params=pltpu.CompilerParams(needs_layout_passes=False),
    scratch_types=[
        pltpu.VMEM((CH, D), jnp.float32),
        pltpu.VMEM((CH, D), jnp.float32),
        pltpu.VMEM((L_SC,), jnp.float32),
        pltpu.VMEM((L_SC,), jnp.float32),
        pltpu.VMEM((D,), jnp.float32),
        pltpu.VMEM((LANES,), jnp.float32),
        pltpu.VMEM((D,), jnp.float32),
        pltpu.VMEM((LANES,), jnp.float32),
        pltpu.SemaphoreType.DMA,
        pltpu.SemaphoreType.DMA,
    ],
)(_sc_body)


def _merge_kernel(o1_ref, m1_ref, l1_ref, sg1_ref, o2_ref, m2_ref, l2_ref,
                  sg2_ref, wval_ref, out_ref):
    m1 = m1_ref[...][:, :1]
    m2 = m2_ref[...][:, :1]
    m = jnp.maximum(m1, m2)
    a1 = jnp.exp(m1 - m)
    a2 = jnp.exp(m2 - m)
    denom = a1 * l1_ref[...][:, :1] + a2 * l2_ref[...][:, :1]
    sgw = a1 * sg1_ref[...][:, :1] + a2 * sg2_ref[...][:, :1]
    out_ref[...] = (a1 * o1_ref[...] + a2 * o2_ref[...]
                    + sgw * wval_ref[...]) / denom


def kernel(s, write_vec, mem_k, mem_v, gate, Wq, Wl, bl, Wk, Wv):
    f32 = jnp.float32
    whole = lambda shape: pl.BlockSpec(shape, lambda g: tuple(0 for _ in shape))
    gw, q, wval, c = pl.pallas_call(
        _pre_kernel,
        out_shape=(
            jax.ShapeDtypeStruct((B, SLOTS), f32),
            jax.ShapeDtypeStruct((B, D), f32),
            jax.ShapeDtypeStruct((B, D), f32),
            jax.ShapeDtypeStruct((B, 128), f32),
        ),
    )(s, write_vec, gate, Wq, Wl, bl, Wk, Wv)

    o2, m2, l2, sg2 = _sc_call(
        mem_k.reshape(B * SLOTS, D),
        mem_v.reshape(B * SLOTS, D),
        gw.reshape(B * SLOTS),
        q.reshape(B * D),
        c.reshape(B * 128),
    )

    o1, m1, l1, sg1 = pl.pallas_call(
        _tc_main,
        grid=(2 * NBT,),
        in_specs=[
            whole((B, D)),
            whole((B, SLOTS)),
            whole((B, 128)),
            pl.BlockSpec((B, SBLK, D),
                         lambda g: (0, jnp.minimum(g, NBT - 1), 0)),
            pl.BlockSpec((B, SBLK, D),
                         lambda g: (0, jnp.maximum(g - NBT, 0), 0)),
        ],
        out_specs=(
            pl.BlockSpec((B, D), lambda g: (0, 0)),
            pl.BlockSpec((B, 128), lambda g: (0, 0)),
            pl.BlockSpec((B, 128), lambda g: (0, 0)),
            pl.BlockSpec((B, 128), lambda g: (0, 0)),
        ),
        out_shape=(
            jax.ShapeDtypeStruct((B, D), f32),
            jax.ShapeDtypeStruct((B, 128), f32),
            jax.ShapeDtypeStruct((B, 128), f32),
            jax.ShapeDtypeStruct((B, 128), f32),
        ),
        scratch_shapes=[
            pltpu.VMEM((B, T), f32),
            pltpu.VMEM((B, D), f32),
        ],
    )(q, gw, c, mem_k, mem_v)

    out = pl.pallas_call(
        _merge_kernel,
        out_shape=jax.ShapeDtypeStruct((B, D), f32),
    )(o1, m1, l1, sg1, o2.reshape(B, D), m2.reshape(B, LANES),
      l2.reshape(B, LANES), sg2.reshape(B, LANES), wval)
    return out


# speedup vs baseline: 4.0536x; 1.0682x over previous
import functools

import jax
import jax.numpy as jnp
from jax import lax
from jax.experimental import pallas as pl
from jax.experimental.pallas import tpu as pltpu
from jax.experimental.pallas import tpu_sc as plsc

B = 32
D = 256
SLOTS = 4096
INV_SQRT_D = 1.0 / 16.0

T = 2304
SBLK = 256
NBT = T // SBLK

NC = 2
NS = 16
LANES = 16
NQ = D // LANES
CH = 128
L_SC = SLOTS - T
NCH = L_SC // CH
GRP = 16
NG_SC = L_SC // LANES


def _pre_kernel(s_ref, wvec_ref, gate_ref, Wq_ref, Wl_ref, bl_ref, Wk_ref,
                Wv_ref, gw_ref, q_ref, wval_ref, c_ref):
    s = s_ref[...]
    logits = jax.lax.dot_general(s, Wl_ref[...], (((1,), (1,)), ((), ())),
                                 preferred_element_type=jnp.float32)
    logits = logits + bl_ref[...][None, :]
    m = jnp.max(logits, axis=-1, keepdims=True)
    e = jnp.exp(logits - m)
    w = e / jnp.sum(e, axis=-1, keepdims=True)
    gw_ref[...] = gate_ref[...] * w
    q = jax.lax.dot_general(s, Wq_ref[...], (((1,), (1,)), ((), ())),
                            preferred_element_type=jnp.float32)
    q_ref[...] = q
    wvec = wvec_ref[...]
    wk = jax.lax.dot_general(wvec, Wk_ref[...], (((1,), (1,)), ((), ())),
                             preferred_element_type=jnp.float32)
    wval_ref[...] = jax.lax.dot_general(wvec, Wv_ref[...],
                                        (((1,), (1,)), ((), ())),
                                        preferred_element_type=jnp.float32)
    c = jnp.sum(q * wk, axis=-1, keepdims=True)
    c_ref[...] = jnp.broadcast_to(c, (B, 128))


def _tc_main(q_ref, gw_ref, c_ref, mk_ref, mv_ref,
             o1_ref, m1_ref, l1_ref, sg1_ref, att_s, acc_s):
    g = pl.program_id(0)

    @pl.when(g < NBT)
    def _att_phase():
        i = g
        q = q_ref[...]
        gw = gw_ref[pl.ds(0, B), pl.ds(i * SBLK, SBLK)]
        c = c_ref[...][:, :1]
        a0 = jnp.sum(q[:, None, :] * mk_ref[...], axis=-1)
        att_s[pl.ds(0, B), pl.ds(i * SBLK, SBLK)] = (
            (a0 * (1.0 - gw) + gw * c) * INV_SQRT_D)

    @pl.when(g == NBT)
    def _softmax_partial():
        att = att_s[...]
        gw = gw_ref[pl.ds(0, B), pl.ds(0, T)]
        m1 = jnp.max(att, axis=-1, keepdims=True)
        e = jnp.exp(att - m1)
        att_s[...] = e * (1.0 - gw)
        m1_ref[...] = jnp.broadcast_to(m1, (B, 128))
        l1_ref[...] = jnp.broadcast_to(
            jnp.sum(e, axis=-1, keepdims=True), (B, 128))
        sg1_ref[...] = jnp.broadcast_to(
            jnp.sum(e * gw, axis=-1, keepdims=True), (B, 128))
        acc_s[...] = jnp.zeros((B, D), jnp.float32)

    @pl.when(g >= NBT)
    def _out_phase():
        i = g - NBT
        wr = att_s[pl.ds(0, B), pl.ds(i * SBLK, SBLK)]
        acc_s[...] += jnp.sum(wr[:, :, None] * mv_ref[...], axis=1)

    @pl.when(g == 2 * NBT - 1)
    def _epilogue():
        o1_ref[...] = acc_s[...]


def _sc_body(memk_ref, memv_ref, gw_ref, q_ref, c_ref,
             o2_ref, m2_ref, l2_ref, sg2_ref,
             kbuf0, kbuf1, gw_v, att_v, q_v, c_v, out_v, stat_v, sem0, sem1):
    cid = lax.axis_index("c")
    sid = lax.axis_index("s")
    wid = sid * NC + cid
    base = wid * SLOTS + T

    pltpu.sync_copy(gw_ref.at[wid, pl.ds(T, L_SC)], gw_v)
    pltpu.sync_copy(q_ref.at[wid], q_v)
    pltpu.sync_copy(c_ref.at[wid, pl.ds(0, LANES)], c_v)

    c_spl = c_v[...]
    lane = lax.iota(jnp.int32, LANES)

    def stream_pass(src_ref, process):
        def start(ci, dst, sem):
            src = src_ref.at[pl.ds(base + ci * CH, CH)]
            pltpu.make_async_copy(src, dst, sem).start()

        def wait(dst, sem):
            src = src_ref.at[pl.ds(base, CH)]
            pltpu.make_async_copy(src, dst, sem).wait()

        start(0, kbuf0, sem0)

        def body(p, _):
            c1 = 2 * p + 1
            start(c1, kbuf1, sem1)
            wait(kbuf0, sem0)
            process(2 * p, kbuf0)

            @pl.when(c1 + 1 < NCH)
            def _():
                start(c1 + 1, kbuf0, sem0)

            wait(kbuf1, sem1)
            process(c1, kbuf1)
            return 0

        lax.fori_loop(0, NCH // 2, body, 0)

    def att_chunk(cidx, buf):
        def grp_body(g, _):
            prod = [jnp.zeros((LANES,), jnp.float32) for _ in range(GRP)]
            for m in range(NQ):
                qm = q_v[pl.ds(m * LANES, LANES)]
                for j in range(GRP):
                    row = g * GRP + j
                    prod[j] = prod[j] + qm * buf[row, pl.ds(m * LANES, LANES)]
            att16 = jnp.zeros((LANES,), jnp.float32)
            for j in range(GRP):
                att16 = jnp.where(lane == j, jnp.sum(prod[j]), att16)
            pos = cidx * CH + g * GRP
            gw16 = gw_v[pl.ds(pos, LANES)]
            att_v[pl.ds(pos, LANES)] = (
                (att16 * (1.0 - gw16) + gw16 * c_spl) * INV_SQRT_D)
            return 0

        lax.fori_loop(0, CH // GRP, grp_body, 0)

    stream_pass(memk_ref, att_chunk)

    def max_body(i, m16):
        return jnp.maximum(m16, att_v[pl.ds(i * LANES, LANES)])

    m16 = lax.fori_loop(0, NG_SC, max_body,
                        jnp.full((LANES,), -1e30, jnp.float32))
    m = jnp.max(m16)

    def exp_body(i, carry):
        s16, sg16 = carry
        a = att_v[pl.ds(i * LANES, LANES)]
        gw16 = gw_v[pl.ds(i * LANES, LANES)]
        e = jnp.exp(a - m)
        att_v[pl.ds(i * LANES, LANES)] = e * (1.0 - gw16)
        return (s16 + e, sg16 + e * gw16)

    zero16 = jnp.zeros((LANES,), jnp.float32)
    s16, sg16 = lax.fori_loop(0, NG_SC, exp_body, (zero16, zero16))
    l2 = jnp.sum(s16)
    sg2 = jnp.sum(sg16)

    for mm in range(NQ):
        out_v[pl.ds(mm * LANES, LANES)] = zero16

    def out_chunk(cidx, buf):
        def grp_body(g, _):
            pos = cidx * CH + g * GRP
            w16 = att_v[pl.ds(pos, LANES)]
            splats = [
                w16.at[jnp.full((LANES,), j, jnp.int32)].get(
                    mode="promise_in_bounds")
                for j in range(GRP)
            ]
            for h in range(2):
                accs = [out_v[pl.ds((8 * h + mm) * LANES, LANES)]
                        for mm in range(8)]
                for j in range(GRP):
                    row = g * GRP + j
                    for mm in range(8):
                        accs[mm] = accs[mm] + splats[j] * \
                            buf[row, pl.ds((8 * h + mm) * LANES, LANES)]
                for mm in range(8):
                    out_v[pl.ds((8 * h + mm) * LANES, LANES)] = accs[mm]
            return 0

        lax.fori_loop(0, CH // GRP, grp_body, 0)

    stream_pass(memv_ref, out_chunk)

    pltpu.sync_copy(out_v, o2_ref.at[wid])
    stat_v[...] = jnp.broadcast_to(m, (LANES,))
    pltpu.sync_copy(stat_v, m2_ref.at[wid])
    stat_v[...] = jnp.broadcast_to(l2, (LANES,))
    pltpu.sync_copy(stat_v, l2_ref.at[wid])
    stat_v[...] = jnp.broadcast_to(sg2, (LANES,))
    pltpu.sync_copy(stat_v, sg2_ref.at[wid])


_sc_call = functools.partial(
    pl.kernel,
    out_type=(
        jax.ShapeDtypeStruct((B, D), jnp.float32),
        jax.ShapeDtypeStruct((B, LANES), jnp.float32),
        jax.ShapeDtypeStruct((B, LANES), jnp.float32),
        jax.ShapeDtypeStruct((B, LANES), jnp.float32),
    ),
    mesh=plsc.VectorSubcoreMesh(core_axis_name="c", subcore_axis_name="s"),
    compiler_params=pltpu.CompilerParams(needs_layout_passes=False),
    scratch_types=[
        pltpu.VMEM((CH, D), jnp.float32),
        pltpu.VMEM((CH, D), jnp.float32),
        pltpu.VMEM((L_SC,), jnp.float32),
        pltpu.VMEM((L_SC,), jnp.float32),
        pltpu.VMEM((D,), jnp.float32),
        pltpu.VMEM((LANES,), jnp.float32),
        pltpu.VMEM((D,), jnp.float32),
        pltpu.VMEM((LANES,), jnp.float32),
        pltpu.SemaphoreType.DMA,
        pltpu.SemaphoreType.DMA,
    ],
)(_sc_body)


def _merge_kernel(o1_ref, m1_ref, l1_ref, sg1_ref, o2_ref, m2_ref, l2_ref,
                  sg2_ref, wval_ref, out_ref):
    m1 = m1_ref[...][:, :1]
    m2 = m2_ref[...][:, :1]
    m = jnp.maximum(m1, m2)
    a1 = jnp.exp(m1 - m)
    a2 = jnp.exp(m2 - m)
    denom = a1 * l1_ref[...][:, :1] + a2 * l2_ref[...][:, :1]
    sgw = a1 * sg1_ref[...][:, :1] + a2 * sg2_ref[...][:, :1]
    out_ref[...] = (a1 * o1_ref[...] + a2 * o2_ref[...]
                    + sgw * wval_ref[...]) / denom


def kernel(s, write_vec, mem_k, mem_v, gate, Wq, Wl, bl, Wk, Wv):
    f32 = jnp.float32
    whole = lambda shape: pl.BlockSpec(shape, lambda g: tuple(0 for _ in shape))
    gw, q, wval, c = pl.pallas_call(
        _pre_kernel,
        out_shape=(
            jax.ShapeDtypeStruct((B, SLOTS), f32),
            jax.ShapeDtypeStruct((B, D), f32),
            jax.ShapeDtypeStruct((B, D), f32),
            jax.ShapeDtypeStruct((B, 128), f32),
        ),
    )(s, write_vec, gate, Wq, Wl, bl, Wk, Wv)

    o2, m2, l2, sg2 = _sc_call(
        mem_k.reshape(B * SLOTS, D),
        mem_v.reshape(B * SLOTS, D),
        gw, q, c,
    )

    o1, m1, l1, sg1 = pl.pallas_call(
        _tc_main,
        grid=(2 * NBT,),
        in_specs=[
            whole((B, D)),
            whole((B, SLOTS)),
            whole((B, 128)),
            pl.BlockSpec((B, SBLK, D),
                         lambda g: (0, jnp.minimum(g, NBT - 1), 0)),
            pl.BlockSpec((B, SBLK, D),
                         lambda g: (0, jnp.maximum(g - NBT, 0), 0)),
        ],
        out_specs=(
            pl.BlockSpec((B, D), lambda g: (0, 0)),
            pl.BlockSpec((B, 128), lambda g: (0, 0)),
            pl.BlockSpec((B, 128), lambda g: (0, 0)),
            pl.BlockSpec((B, 128), lambda g: (0, 0)),
        ),
        out_shape=(
            jax.ShapeDtypeStruct((B, D), f32),
            jax.ShapeDtypeStruct((B, 128), f32),
            jax.ShapeDtypeStruct((B, 128), f32),
            jax.ShapeDtypeStruct((B, 128), f32),
        ),
        scratch_shapes=[
            pltpu.VMEM((B, T), f32),
            pltpu.VMEM((B, D), f32),
        ],
    )(q, gw, c, mem_k, mem_v)

    out = pl.pallas_call(
        _merge_kernel,
        out_shape=jax.ShapeDtypeStruct((B, D), f32),
    )(o1, m1, l1, sg1, o2, m2, l2, sg2, wval)
    return out


# speedup vs baseline: 4.1090x; 1.0137x over previous
import functools

import jax
import jax.numpy as jnp
from jax import lax
from jax.experimental import pallas as pl
from jax.experimental.pallas import tpu as pltpu
from jax.experimental.pallas import tpu_sc as plsc

B = 32
D = 256
SLOTS = 4096
INV_SQRT_D = 1.0 / 16.0

T = 2560
SBLK = 256
NBT = T // SBLK

NC = 2
NS = 16
LANES = 16
NQ = D // LANES
CH = 128
L_SC = SLOTS - T
NCH = L_SC // CH
GRP = 16
NG_SC = L_SC // LANES


def _pre_kernel(s_ref, wvec_ref, gate_ref, Wq_ref, Wl_ref, bl_ref, Wk_ref,
                Wv_ref, gw_ref, q_ref, wval_ref, c_ref):
    s = s_ref[...]
    logits = jax.lax.dot_general(s, Wl_ref[...], (((1,), (1,)), ((), ())),
                                 preferred_element_type=jnp.float32)
    logits = logits + bl_ref[...][None, :]
    m = jnp.max(logits, axis=-1, keepdims=True)
    e = jnp.exp(logits - m)
    w = e / jnp.sum(e, axis=-1, keepdims=True)
    gw_ref[...] = gate_ref[...] * w
    q = jax.lax.dot_general(s, Wq_ref[...], (((1,), (1,)), ((), ())),
                            preferred_element_type=jnp.float32)
    q_ref[...] = q
    wvec = wvec_ref[...]
    wk = jax.lax.dot_general(wvec, Wk_ref[...], (((1,), (1,)), ((), ())),
                             preferred_element_type=jnp.float32)
    wval_ref[...] = jax.lax.dot_general(wvec, Wv_ref[...],
                                        (((1,), (1,)), ((), ())),
                                        preferred_element_type=jnp.float32)
    c = jnp.sum(q * wk, axis=-1, keepdims=True)
    c_ref[...] = jnp.broadcast_to(c, (B, 128))


def _tc_main(q_ref, gw_ref, c_ref, mk_ref, mv_ref,
             o1_ref, m1_ref, l1_ref, sg1_ref, att_s, acc_s):
    g = pl.program_id(0)

    @pl.when(g < NBT)
    def _att_phase():
        i = g
        q = q_ref[...]
        gw = gw_ref[pl.ds(0, B), pl.ds(i * SBLK, SBLK)]
        c = c_ref[...][:, :1]
        a0 = jnp.sum(q[:, None, :] * mk_ref[...], axis=-1)
        att_s[pl.ds(0, B), pl.ds(i * SBLK, SBLK)] = (
            (a0 * (1.0 - gw) + gw * c) * INV_SQRT_D)

    @pl.when(g == NBT)
    def _softmax_partial():
        att = att_s[...]
        gw = gw_ref[pl.ds(0, B), pl.ds(0, T)]
        m1 = jnp.max(att, axis=-1, keepdims=True)
        e = jnp.exp(att - m1)
        att_s[...] = e * (1.0 - gw)
        m1_ref[...] = jnp.broadcast_to(m1, (B, 128))
        l1_ref[...] = jnp.broadcast_to(
            jnp.sum(e, axis=-1, keepdims=True), (B, 128))
        sg1_ref[...] = jnp.broadcast_to(
            jnp.sum(e * gw, axis=-1, keepdims=True), (B, 128))
        acc_s[...] = jnp.zeros((B, D), jnp.float32)

    @pl.when(g >= NBT)
    def _out_phase():
        i = g - NBT
        wr = att_s[pl.ds(0, B), pl.ds(i * SBLK, SBLK)]
        acc_s[...] += jnp.sum(wr[:, :, None] * mv_ref[...], axis=1)

    @pl.when(g == 2 * NBT - 1)
    def _epilogue():
        o1_ref[...] = acc_s[...]


def _sc_body(memk_ref, memv_ref, gw_ref, q_ref, c_ref,
             o2_ref, m2_ref, l2_ref, sg2_ref,
             kbuf0, kbuf1, gw_v, att_v, q_v, c_v, out_v, stat_v, sem0, sem1):
    cid = lax.axis_index("c")
    sid = lax.axis_index("s")
    wid = sid * NC + cid
    base = wid * SLOTS + T

    pltpu.sync_copy(gw_ref.at[wid, pl.ds(T, L_SC)], gw_v)
    pltpu.sync_copy(q_ref.at[wid], q_v)
    pltpu.sync_copy(c_ref.at[wid, pl.ds(0, LANES)], c_v)

    c_spl = c_v[...]
    lane = lax.iota(jnp.int32, LANES)

    def stream_pass(src_ref, process):
        def start(ci, dst, sem):
            src = src_ref.at[pl.ds(base + ci * CH, CH)]
            pltpu.make_async_copy(src, dst, sem).start()

        def wait(dst, sem):
            src = src_ref.at[pl.ds(base, CH)]
            pltpu.make_async_copy(src, dst, sem).wait()

        start(0, kbuf0, sem0)

        def body(p, _):
            c1 = 2 * p + 1
            start(c1, kbuf1, sem1)
            wait(kbuf0, sem0)
            process(2 * p, kbuf0)

            @pl.when(c1 + 1 < NCH)
            def _():
                start(c1 + 1, kbuf0, sem0)

            wait(kbuf1, sem1)
            process(c1, kbuf1)
            return 0

        lax.fori_loop(0, NCH // 2, body, 0)

    def att_chunk(cidx, buf):
        def grp_body(g, _):
            prod = [jnp.zeros((LANES,), jnp.float32) for _ in range(GRP)]
            for m in range(NQ):
                qm = q_v[pl.ds(m * LANES, LANES)]
                for j in range(GRP):
                    row = g * GRP + j
                    prod[j] = prod[j] + qm * buf[row, pl.ds(m * LANES, LANES)]
            att16 = jnp.zeros((LANES,), jnp.float32)
            for j in range(GRP):
                att16 = jnp.where(lane == j, jnp.sum(prod[j]), att16)
            pos = cidx * CH + g * GRP
            gw16 = gw_v[pl.ds(pos, LANES)]
            att_v[pl.ds(pos, LANES)] = (
                (att16 * (1.0 - gw16) + gw16 * c_spl) * INV_SQRT_D)
            return 0

        lax.fori_loop(0, CH // GRP, grp_body, 0)

    stream_pass(memk_ref, att_chunk)

    def max_body(i, m16):
        return jnp.maximum(m16, att_v[pl.ds(i * LANES, LANES)])

    m16 = lax.fori_loop(0, NG_SC, max_body,
                        jnp.full((LANES,), -1e30, jnp.float32))
    m = jnp.max(m16)

    def exp_body(i, carry):
        s16, sg16 = carry
        a = att_v[pl.ds(i * LANES, LANES)]
        gw16 = gw_v[pl.ds(i * LANES, LANES)]
        e = jnp.exp(a - m)
        att_v[pl.ds(i * LANES, LANES)] = e * (1.0 - gw16)
        return (s16 + e, sg16 + e * gw16)

    zero16 = jnp.zeros((LANES,), jnp.float32)
    s16, sg16 = lax.fori_loop(0, NG_SC, exp_body, (zero16, zero16))
    l2 = jnp.sum(s16)
    sg2 = jnp.sum(sg16)

    for mm in range(NQ):
        out_v[pl.ds(mm * LANES, LANES)] = zero16

    def out_chunk(cidx, buf):
        def grp_body(g, _):
            pos = cidx * CH + g * GRP
            w16 = att_v[pl.ds(pos, LANES)]
            splats = [
                w16.at[jnp.full((LANES,), j, jnp.int32)].get(
                    mode="promise_in_bounds")
                for j in range(GRP)
            ]
            for h in range(2):
                accs = [out_v[pl.ds((8 * h + mm) * LANES, LANES)]
                        for mm in range(8)]
                for j in range(GRP):
                    row = g * GRP + j
                    for mm in range(8):
                        accs[mm] = accs[mm] + splats[j] * \
                            buf[row, pl.ds((8 * h + mm) * LANES, LANES)]
                for mm in range(8):
                    out_v[pl.ds((8 * h + mm) * LANES, LANES)] = accs[mm]
            return 0

        lax.fori_loop(0, CH // GRP, grp_body, 0)

    stream_pass(memv_ref, out_chunk)

    pltpu.sync_copy(out_v, o2_ref.at[wid])
    stat_v[...] = jnp.broadcast_to(m, (LANES,))
    pltpu.sync_copy(stat_v, m2_ref.at[wid])
    stat_v[...] = jnp.broadcast_to(l2, (LANES,))
    pltpu.sync_copy(stat_v, l2_ref.at[wid])
    stat_v[...] = jnp.broadcast_to(sg2, (LANES,))
    pltpu.sync_copy(stat_v, sg2_ref.at[wid])


_sc_call = functools.partial(
    pl.kernel,
    out_type=(
        jax.ShapeDtypeStruct((B, D), jnp.float32),
        jax.ShapeDtypeStruct((B, LANES), jnp.float32),
        jax.ShapeDtypeStruct((B, LANES), jnp.float32),
        jax.ShapeDtypeStruct((B, LANES), jnp.float32),
    ),
    mesh=plsc.VectorSubcoreMesh(core_axis_name="c", subcore_axis_name="s"),
    compiler_params=pltpu.CompilerParams(needs_layout_passes=False),
    scratch_types=[
        pltpu.VMEM((CH, D), jnp.float32),
        pltpu.VMEM((CH, D), jnp.float32),
        pltpu.VMEM((L_SC,), jnp.float32),
        pltpu.VMEM((L_SC,), jnp.float32),
        pltpu.VMEM((D,), jnp.float32),
        pltpu.VMEM((LANES,), jnp.float32),
        pltpu.VMEM((D,), jnp.float32),
        pltpu.VMEM((LANES,), jnp.float32),
        pltpu.SemaphoreType.DMA,
        pltpu.SemaphoreType.DMA,
    ],
)(_sc_body)


def _merge_kernel(o1_ref, m1_ref, l1_ref, sg1_ref, o2_ref, m2_ref, l2_ref,
                  sg2_ref, wval_ref, out_ref):
    m1 = m1_ref[...][:, :1]
    m2 = m2_ref[...][:, :1]
    m = jnp.maximum(m1, m2)
    a1 = jnp.exp(m1 - m)
    a2 = jnp.exp(m2 - m)
    denom = a1 * l1_ref[...][:, :1] + a2 * l2_ref[...][:, :1]
    sgw = a1 * sg1_ref[...][:, :1] + a2 * sg2_ref[...][:, :1]
    out_ref[...] = (a1 * o1_ref[...] + a2 * o2_ref[...]
                    + sgw * wval_ref[...]) / denom


def kernel(s, write_vec, mem_k, mem_v, gate, Wq, Wl, bl, Wk, Wv):
    f32 = jnp.float32
    whole = lambda shape: pl.BlockSpec(shape, lambda g: tuple(0 for _ in shape))
    gw, q, wval, c = pl.pallas_call(
        _pre_kernel,
        out_shape=(
            jax.ShapeDtypeStruct((B, SLOTS), f32),
            jax.ShapeDtypeStruct((B, D), f32),
            jax.ShapeDtypeStruct((B, D), f32),
            jax.ShapeDtypeStruct((B, 128), f32),
        ),
    )(s, write_vec, gate, Wq, Wl, bl, Wk, Wv)

    o2, m2, l2, sg2 = _sc_call(
        mem_k.reshape(B * SLOTS, D),
        mem_v.reshape(B * SLOTS, D),
        gw, q, c,
    )

    o1, m1, l1, sg1 = pl.pallas_call(
        _tc_main,
        grid=(2 * NBT,),
        in_specs=[
            whole((B, D)),
            whole((B, SLOTS)),
            whole((B, 128)),
            pl.BlockSpec((B, SBLK, D),
                         lambda g: (0, jnp.minimum(g, NBT - 1), 0)),
            pl.BlockSpec((B, SBLK, D),
                         lambda g: (0, jnp.maximum(g - NBT, 0), 0)),
        ],
        out_specs=(
            pl.BlockSpec((B, D), lambda g: (0, 0)),
            pl.BlockSpec((B, 128), lambda g: (0, 0)),
            pl.BlockSpec((B, 128), lambda g: (0, 0)),
            pl.BlockSpec((B, 128), lambda g: (0, 0)),
        ),
        out_shape=(
            jax.ShapeDtypeStruct((B, D), f32),
            jax.ShapeDtypeStruct((B, 128), f32),
            jax.ShapeDtypeStruct((B, 128), f32),
            jax.ShapeDtypeStruct((B, 128), f32),
        ),
        scratch_shapes=[
            pltpu.VMEM((B, T), f32),
            pltpu.VMEM((B, D), f32),
        ],
    )(q, gw, c, mem_k, mem_v)

    out = pl.pallas_call(
        _merge_kernel,
        out_shape=jax.ShapeDtypeStruct((B, D), f32),
    )(o1, m1, l1, sg1, o2, m2, l2, sg2, wval)
    return out
